# Initial kernel scaffold; baseline (speedup 1.0000x reference)
import jax
import jax.numpy as jnp
from jax.experimental import pallas as pl


def kernel(x, codebook):
    raise NotImplementedError("write your pallas kernel here")


# speedup vs baseline: 1.3675x; 1.3675x over previous
import functools

import jax
import jax.numpy as jnp
from jax import lax
from jax.experimental import pallas as pl
from jax.experimental.pallas import tpu as pltpu
from jax.experimental.pallas import tpu_sc as plsc

TILE_I = 1024
TILE_J = 1024


def _dist_argmin_body(x2_ref, x_ref, cb_ref, enc_ref, loss_ref,
                      best_ref, bidx_ref):
    i = pl.program_id(0)
    j = pl.program_id(1)
    nj = pl.num_programs(1)

    @pl.when(j == 0)
    def _init():
        best_ref[...] = jnp.full(best_ref.shape, jnp.inf, best_ref.dtype)
        bidx_ref[...] = jnp.zeros(bidx_ref.shape, bidx_ref.dtype)

    x = x_ref[...]
    cb = cb_ref[...]
    m = lax.dot_general(cb, x, (((1,), (1,)), ((), ())),
                        preferred_element_type=jnp.float32)
    c2 = jnp.sum(cb * cb, axis=1, keepdims=True)
    d2 = (x2_ref[...] + c2) - 2.0 * m
    d2 = jnp.maximum(d2, 0.0)
    tmin = jnp.min(d2, axis=0, keepdims=True)
    jj = lax.broadcasted_iota(jnp.int32, d2.shape, 0)
    targ = jnp.min(jnp.where(d2 == tmin, jj, jnp.iinfo(jnp.int32).max),
                   axis=0, keepdims=True) + j * TILE_J
    better = tmin < best_ref[...]
    bidx_ref[...] = jnp.where(better, targ, bidx_ref[...])
    best_ref[...] = jnp.where(better, tmin, best_ref[...])

    @pl.when(j == nj - 1)
    def _fin():
        enc_ref[...] = bidx_ref[...].reshape(enc_ref.shape)
        part = jnp.sum(best_ref[...])
        prev = jnp.where(i == 0, jnp.float32(0.0), loss_ref[0, 0])
        loss_ref[0, 0] = prev + part


def _dist_argmin(xf, codebook, x2):
    M, K = xf.shape
    N = codebook.shape[0]
    ni, nj = M // TILE_I, N // TILE_J
    enc3, loss = pl.pallas_call(
        _dist_argmin_body,
        grid=(ni, nj),
        in_specs=[
            pl.BlockSpec((1, TILE_I), lambda i, j: (0, i)),
            pl.BlockSpec((TILE_I, K), lambda i, j: (i, 0)),
            pl.BlockSpec((TILE_J, K), lambda i, j: (j, 0)),
        ],
        out_specs=[
            pl.BlockSpec((1, 1, TILE_I), lambda i, j: (i, 0, 0)),
            pl.BlockSpec(memory_space=pltpu.SMEM),
        ],
        out_shape=[
            jax.ShapeDtypeStruct((ni, 1, TILE_I), jnp.int32),
            jax.ShapeDtypeStruct((1, 1), jnp.float32),
        ],
        scratch_shapes=[
            pltpu.VMEM((1, TILE_I), jnp.float32),
            pltpu.VMEM((1, TILE_I), jnp.int32),
        ],
    )(x2, xf, codebook)
    return enc3.reshape(M), loss[0, 0]


def _sc_gather(table, idx):
    info = plsc.get_sparse_core_info()
    nw = info.num_cores * info.num_subcores
    B = idx.shape[0]
    D = table.shape[1]
    bpw = B // nw
    mesh = plsc.VectorSubcoreMesh(core_axis_name="c", subcore_axis_name="s")

    @functools.partial(
        pl.kernel, mesh=mesh,
        out_type=jax.ShapeDtypeStruct((B, D), table.dtype),
        scratch_types=[
            pltpu.VMEM((bpw,), jnp.int32),
            pltpu.VMEM((bpw, D), jnp.float32),
            pltpu.SemaphoreType.DMA,
        ],
    )
    def gk(table_hbm, idx_hbm, out_hbm, idx_v, rows_v, sem):
        wid = lax.axis_index("s") * info.num_cores + lax.axis_index("c")
        base = wid * bpw
        pltpu.sync_copy(idx_hbm.at[pl.ds(base, bpw)], idx_v)
        pltpu.async_copy(table_hbm.at[idx_v], rows_v, sem).wait()
        pltpu.sync_copy(rows_v, out_hbm.at[pl.ds(base, bpw)])

    return gk(table, idx)


def kernel(x, codebook):
    K = x.shape[-1]
    xf = x.reshape(-1, K)
    x2 = jnp.sum(xf * xf, axis=1)[None, :]
    encoding, loss_sum = _dist_argmin(xf, codebook, x2)
    nearest = _sc_gather(codebook, encoding)
    loss = loss_sum / jnp.float32(xf.shape[0] * K)
    return encoding, loss, loss, nearest


# speedup vs baseline: 1.4590x; 1.0669x over previous
import functools

import jax
import jax.numpy as jnp
from jax import lax
from jax.experimental import pallas as pl
from jax.experimental.pallas import tpu as pltpu
from jax.experimental.pallas import tpu_sc as plsc

TILE_I = 1024
TILE_J = 1024


def _dist_argmin_body(x2_ref, x_ref, cb_ref, enc_ref, loss_ref,
                      best_ref, bidx_ref):
    i = pl.program_id(0)
    j = pl.program_id(1)
    nj = pl.num_programs(1)

    @pl.when(j == 0)
    def _init():
        best_ref[...] = jnp.full(best_ref.shape, jnp.inf, best_ref.dtype)
        bidx_ref[...] = jnp.zeros(bidx_ref.shape, bidx_ref.dtype)

    x = x_ref[...]
    cb = cb_ref[...]
    m = lax.dot_general(cb, x, (((1,), (1,)), ((), ())),
                        preferred_element_type=jnp.float32)
    c2 = jnp.sum(cb * cb, axis=1, keepdims=True)
    d2 = (x2_ref[...] + c2) + m
    tmin_raw = jnp.min(d2, axis=0, keepdims=True)
    jj = lax.broadcasted_iota(jnp.int32, d2.shape, 0).astype(jnp.float32)
    targ_f = jnp.min(jnp.where(d2 == tmin_raw, jj, jnp.inf),
                     axis=0, keepdims=True)
    targ = targ_f.astype(jnp.int32) + j * TILE_J
    tmin = jnp.maximum(tmin_raw, 0.0)
    better = tmin < best_ref[...]
    bidx_ref[...] = jnp.where(better, targ, bidx_ref[...])
    best_ref[...] = jnp.where(better, tmin, best_ref[...])

    @pl.when(j == nj - 1)
    def _fin():
        enc_ref[...] = bidx_ref[...].reshape(enc_ref.shape)
        part = jnp.sum(best_ref[...])
        prev = jnp.where(i == 0, jnp.float32(0.0), loss_ref[0, 0])
        loss_ref[0, 0] = prev + part


def _dist_argmin(xf, codebook, x2):
    M, K = xf.shape
    N = codebook.shape[0]
    ni, nj = M // TILE_I, N // TILE_J
    enc3, loss = pl.pallas_call(
        _dist_argmin_body,
        grid=(ni, nj),
        in_specs=[
            pl.BlockSpec((1, TILE_I), lambda i, j: (0, i)),
            pl.BlockSpec((TILE_I, K), lambda i, j: (i, 0)),
            pl.BlockSpec((TILE_J, K), lambda i, j: (j, 0)),
        ],
        out_specs=[
            pl.BlockSpec((1, 1, TILE_I), lambda i, j: (i, 0, 0)),
            pl.BlockSpec(memory_space=pltpu.SMEM),
        ],
        out_shape=[
            jax.ShapeDtypeStruct((ni, 1, TILE_I), jnp.int32),
            jax.ShapeDtypeStruct((1, 1), jnp.float32),
        ],
        scratch_shapes=[
            pltpu.VMEM((1, TILE_I), jnp.float32),
            pltpu.VMEM((1, TILE_I), jnp.int32),
        ],
        compiler_params=pltpu.CompilerParams(
            dimension_semantics=("parallel", "arbitrary")),
    )(x2, xf, codebook)
    return enc3.reshape(M), loss[0, 0]


def _sc_gather(table, idx):
    info = plsc.get_sparse_core_info()
    nw = info.num_cores * info.num_subcores
    B = idx.shape[0]
    D = table.shape[1]
    bpw = B // nw
    mesh = plsc.VectorSubcoreMesh(core_axis_name="c", subcore_axis_name="s")

    @functools.partial(
        pl.kernel, mesh=mesh,
        out_type=jax.ShapeDtypeStruct((B, D), table.dtype),
        scratch_types=[
            pltpu.VMEM((bpw,), jnp.int32),
            pltpu.VMEM((bpw, D), jnp.float32),
            pltpu.SemaphoreType.DMA,
        ],
    )
    def gk(table_hbm, idx_hbm, out_hbm, idx_v, rows_v, sem):
        wid = lax.axis_index("s") * info.num_cores + lax.axis_index("c")
        base = wid * bpw
        pltpu.sync_copy(idx_hbm.at[pl.ds(base, bpw)], idx_v)
        pltpu.async_copy(table_hbm.at[idx_v], rows_v, sem).wait()
        pltpu.sync_copy(rows_v, out_hbm.at[pl.ds(base, bpw)])

    return gk(table, idx)


def kernel(x, codebook):
    K = x.shape[-1]
    xf = x.reshape(-1, K)
    x2 = jnp.sum(xf * xf, axis=1)[None, :]
    xs = xf * jnp.float32(-2.0)
    encoding, loss_sum = _dist_argmin(xs, codebook, x2)
    nearest = _sc_gather(codebook, encoding)
    loss = loss_sum / jnp.float32(xf.shape[0] * K)
    return encoding, loss, loss, nearest


# speedup vs baseline: 1.6402x; 1.1242x over previous
import functools

import jax
import jax.numpy as jnp
from jax import lax
from jax.experimental import pallas as pl
from jax.experimental.pallas import tpu as pltpu
from jax.experimental.pallas import tpu_sc as plsc

TILE_I = 1024
TILE_J = 1024


def _dist_argmin_body(x2_ref, x_ref, cb_ref, enc_ref, loss_ref,
                      best_ref, bidx_ref):
    i = pl.program_id(0)
    j = pl.program_id(1)
    nj = pl.num_programs(1)

    @pl.when(j == 0)
    def _init():
        best_ref[...] = jnp.full(best_ref.shape, jnp.inf, best_ref.dtype)
        bidx_ref[...] = jnp.zeros(bidx_ref.shape, bidx_ref.dtype)

    x = x_ref[...]
    cb = cb_ref[...]
    m = lax.dot_general(cb, x, (((1,), (1,)), ((), ())),
                        preferred_element_type=jnp.float32)
    c2 = jnp.sum(cb * cb, axis=1, keepdims=True)
    d2 = (x2_ref[...] + c2) + m
    tmin_raw = jnp.min(d2, axis=0, keepdims=True)
    targ = (jnp.argmin(d2, axis=0).astype(jnp.int32).reshape(1, TILE_I)
            + j * TILE_J)
    tmin = jnp.maximum(tmin_raw, 0.0)
    better = tmin < best_ref[...]
    bidx_ref[...] = jnp.where(better, targ, bidx_ref[...])
    best_ref[...] = jnp.where(better, tmin, best_ref[...])

    @pl.when(j == nj - 1)
    def _fin():
        enc_ref[...] = bidx_ref[...].reshape(enc_ref.shape)
        part = jnp.sum(best_ref[...])
        prev = jnp.where(i == 0, jnp.float32(0.0), loss_ref[0, 0])
        loss_ref[0, 0] = prev + part


def _dist_argmin(xf, codebook, x2):
    M, K = xf.shape
    N = codebook.shape[0]
    ni, nj = M // TILE_I, N // TILE_J
    enc3, loss = pl.pallas_call(
        _dist_argmin_body,
        grid=(ni, nj),
        in_specs=[
            pl.BlockSpec((1, TILE_I), lambda i, j: (0, i)),
            pl.BlockSpec((TILE_I, K), lambda i, j: (i, 0)),
            pl.BlockSpec((TILE_J, K), lambda i, j: (j, 0)),
        ],
        out_specs=[
            pl.BlockSpec((1, 1, TILE_I), lambda i, j: (i, 0, 0)),
            pl.BlockSpec(memory_space=pltpu.SMEM),
        ],
        out_shape=[
            jax.ShapeDtypeStruct((ni, 1, TILE_I), jnp.int32),
            jax.ShapeDtypeStruct((1, 1), jnp.float32),
        ],
        scratch_shapes=[
            pltpu.VMEM((1, TILE_I), jnp.float32),
            pltpu.VMEM((1, TILE_I), jnp.int32),
        ],
        compiler_params=pltpu.CompilerParams(
            dimension_semantics=("parallel", "arbitrary")),
    )(x2, xf, codebook)
    return enc3.reshape(M), loss[0, 0]


def _sc_gather(table, idx):
    info = plsc.get_sparse_core_info()
    nw = info.num_cores * info.num_subcores
    B = idx.shape[0]
    D = table.shape[1]
    bpw = B // nw
    mesh = plsc.VectorSubcoreMesh(core_axis_name="c", subcore_axis_name="s")

    @functools.partial(
        pl.kernel, mesh=mesh,
        out_type=jax.ShapeDtypeStruct((B, D), table.dtype),
        scratch_types=[
            pltpu.VMEM((bpw,), jnp.int32),
            pltpu.VMEM((bpw, D), jnp.float32),
            pltpu.SemaphoreType.DMA,
        ],
    )
    def gk(table_hbm, idx_hbm, out_hbm, idx_v, rows_v, sem):
        wid = lax.axis_index("s") * info.num_cores + lax.axis_index("c")
        base = wid * bpw
        pltpu.sync_copy(idx_hbm.at[pl.ds(base, bpw)], idx_v)
        pltpu.async_copy(table_hbm.at[idx_v], rows_v, sem).wait()
        pltpu.sync_copy(rows_v, out_hbm.at[pl.ds(base, bpw)])

    return gk(table, idx)


def kernel(x, codebook):
    K = x.shape[-1]
    xf = x.reshape(-1, K)
    x2 = jnp.sum(xf * xf, axis=1)[None, :]
    xs = xf * jnp.float32(-2.0)
    encoding, loss_sum = _dist_argmin(xs, codebook, x2)
    nearest = _sc_gather(codebook, encoding)
    loss = loss_sum / jnp.float32(xf.shape[0] * K)
    return encoding, loss, loss, nearest


# speedup vs baseline: 1.9105x; 1.1648x over previous
import functools

import jax
import jax.numpy as jnp
from jax import lax
from jax.experimental import pallas as pl
from jax.experimental.pallas import tpu as pltpu
from jax.experimental.pallas import tpu_sc as plsc

TILE_I = 4096
TILE_J = 2048


def _dist_argmin_body(x2_ref, x_ref, cb_ref, enc_ref, loss_ref,
                      best_ref, bidx_ref, *, inv_n):
    i = pl.program_id(0)
    j = pl.program_id(1)
    ni = pl.num_programs(0)
    nj = pl.num_programs(1)

    @pl.when(j == 0)
    def _init():
        best_ref[...] = jnp.full(best_ref.shape, jnp.inf, best_ref.dtype)
        bidx_ref[...] = jnp.zeros(bidx_ref.shape, bidx_ref.dtype)

    x = x_ref[...] * jnp.float32(-2.0)
    cb = cb_ref[...]
    m = lax.dot_general(cb, x, (((1,), (1,)), ((), ())),
                        preferred_element_type=jnp.float32)
    c2 = jnp.sum(cb * cb, axis=1, keepdims=True)
    d2 = (x2_ref[...] + c2) + m
    tmin_raw = jnp.min(d2, axis=0, keepdims=True)
    targ = (jnp.argmin(d2, axis=0).astype(jnp.int32).reshape(1, TILE_I)
            + j * TILE_J)
    tmin = jnp.maximum(tmin_raw, 0.0)
    better = tmin < best_ref[...]
    bidx_ref[...] = jnp.where(better, targ, bidx_ref[...])
    best_ref[...] = jnp.where(better, tmin, best_ref[...])

    @pl.when(j == nj - 1)
    def _fin():
        enc_ref[...] = bidx_ref[...].reshape(enc_ref.shape)
        part = jnp.sum(best_ref[...])
        prev = jnp.where(i == 0, jnp.float32(0.0), loss_ref[0, 0])
        tot = prev + part
        loss_ref[0, 0] = jnp.where(i == ni - 1, tot * inv_n, tot)


def _dist_argmin(xf, codebook, x2):
    M, K = xf.shape
    N = codebook.shape[0]
    ni, nj = M // TILE_I, N // TILE_J
    enc3, loss = pl.pallas_call(
        functools.partial(_dist_argmin_body, inv_n=1.0 / (M * K)),
        grid=(ni, nj),
        in_specs=[
            pl.BlockSpec((1, TILE_I), lambda i, j: (0, i)),
            pl.BlockSpec((TILE_I, K), lambda i, j: (i, 0)),
            pl.BlockSpec((TILE_J, K), lambda i, j: (j, 0)),
        ],
        out_specs=[
            pl.BlockSpec((1, 1, TILE_I), lambda i, j: (i, 0, 0)),
            pl.BlockSpec(memory_space=pltpu.SMEM),
        ],
        out_shape=[
            jax.ShapeDtypeStruct((ni, 1, TILE_I), jnp.int32),
            jax.ShapeDtypeStruct((1, 1), jnp.float32),
        ],
        scratch_shapes=[
            pltpu.VMEM((1, TILE_I), jnp.float32),
            pltpu.VMEM((1, TILE_I), jnp.int32),
        ],
        compiler_params=pltpu.CompilerParams(
            dimension_semantics=("parallel", "arbitrary")),
    )(x2, xf, codebook)
    return enc3.reshape(M), loss[0, 0]


def _sc_gather(table, idx):
    info = plsc.get_sparse_core_info()
    nw = info.num_cores * info.num_subcores
    B = idx.shape[0]
    D = table.shape[1]
    bpw = B // nw
    mesh = plsc.VectorSubcoreMesh(core_axis_name="c", subcore_axis_name="s")

    half = bpw // 2

    @functools.partial(
        pl.kernel, mesh=mesh,
        out_type=jax.ShapeDtypeStruct((B, D), table.dtype),
        scratch_types=[
            pltpu.VMEM((half,), jnp.int32),
            pltpu.VMEM((half,), jnp.int32),
            pltpu.VMEM((half, D), jnp.float32),
            pltpu.VMEM((half, D), jnp.float32),
            pltpu.SemaphoreType.DMA,
            pltpu.SemaphoreType.DMA,
            pltpu.SemaphoreType.DMA,
        ],
    )
    def gk(table_hbm, idx_hbm, out_hbm, idx0, idx1, rows0, rows1,
           sg0, sg1, sw):
        wid = lax.axis_index("s") * info.num_cores + lax.axis_index("c")
        base = wid * bpw
        pltpu.sync_copy(idx_hbm.at[pl.ds(base, half)], idx0)
        g0 = pltpu.async_copy(table_hbm.at[idx0], rows0, sg0)
        pltpu.sync_copy(idx_hbm.at[pl.ds(base + half, half)], idx1)
        g1 = pltpu.async_copy(table_hbm.at[idx1], rows1, sg1)
        g0.wait()
        w0 = pltpu.async_copy(rows0, out_hbm.at[pl.ds(base, half)], sw)
        g1.wait()
        w1 = pltpu.async_copy(rows1, out_hbm.at[pl.ds(base + half, half)], sw)
        w0.wait()
        w1.wait()

    return gk(table, idx)


def kernel(x, codebook):
    K = x.shape[-1]
    xf = x.reshape(-1, K)
    x2 = jnp.sum(xf * xf, axis=1)[None, :]
    encoding, loss = _dist_argmin(xf, codebook, x2)
    nearest = _sc_gather(codebook, encoding)
    return encoding, loss, loss, nearest


# speedup vs baseline: 1.9308x; 1.0106x over previous
import functools

import jax
import jax.numpy as jnp
from jax import lax
from jax.experimental import pallas as pl
from jax.experimental.pallas import tpu as pltpu
from jax.experimental.pallas import tpu_sc as plsc

TILE_I = 4096
TILE_J = 2048


def _dist_argmin_body(x2_ref, x_ref, cb_ref, enc_ref, loss_ref,
                      best_ref, bidx_ref, *, inv_n):
    i = pl.program_id(0)
    j = pl.program_id(1)
    ni = pl.num_programs(0)
    nj = pl.num_programs(1)

    @pl.when(j == 0)
    def _init():
        best_ref[...] = jnp.full(best_ref.shape, jnp.inf, best_ref.dtype)
        bidx_ref[...] = jnp.zeros(bidx_ref.shape, bidx_ref.dtype)

    x = x_ref[...] * jnp.float32(-2.0)
    cb = cb_ref[...]
    m = lax.dot_general(cb, x, (((1,), (1,)), ((), ())),
                        preferred_element_type=jnp.float32)
    c2 = jnp.sum(cb * cb, axis=1, keepdims=True)
    d2 = (x2_ref[...] + c2) + m
    tmin_raw = jnp.min(d2, axis=0, keepdims=True)
    targ = (jnp.argmin(d2, axis=0).astype(jnp.int32).reshape(1, TILE_I)
            + j * TILE_J)
    tmin = jnp.maximum(tmin_raw, 0.0)
    better = tmin < best_ref[...]
    bidx_ref[...] = jnp.where(better, targ, bidx_ref[...])
    best_ref[...] = jnp.where(better, tmin, best_ref[...])

    @pl.when(j == nj - 1)
    def _fin():
        enc_ref[...] = bidx_ref[...].reshape(enc_ref.shape)
        part = jnp.sum(best_ref[...])
        prev = jnp.where(i == 0, jnp.float32(0.0), loss_ref[0, 0])
        tot = prev + part
        loss_ref[0, 0] = jnp.where(i == ni - 1, tot * inv_n, tot)


def _dist_argmin(xf, codebook, x2):
    M, K = xf.shape
    N = codebook.shape[0]
    ni, nj = M // TILE_I, N // TILE_J
    enc3, loss = pl.pallas_call(
        functools.partial(_dist_argmin_body, inv_n=1.0 / (M * K)),
        grid=(ni, nj),
        in_specs=[
            pl.BlockSpec((1, TILE_I), lambda i, j: (0, i)),
            pl.BlockSpec((TILE_I, K), lambda i, j: (i, 0)),
            pl.BlockSpec((TILE_J, K), lambda i, j: (j, 0)),
        ],
        out_specs=[
            pl.BlockSpec((1, 1, TILE_I), lambda i, j: (i, 0, 0)),
            pl.BlockSpec(memory_space=pltpu.SMEM),
        ],
        out_shape=[
            jax.ShapeDtypeStruct((ni, 1, TILE_I), jnp.int32),
            jax.ShapeDtypeStruct((1, 1), jnp.float32),
        ],
        scratch_shapes=[
            pltpu.VMEM((1, TILE_I), jnp.float32),
            pltpu.VMEM((1, TILE_I), jnp.int32),
        ],
        compiler_params=pltpu.CompilerParams(
            dimension_semantics=("parallel", "arbitrary")),
    )(x2, xf, codebook)
    return enc3.reshape(M), loss[0, 0]


def _sc_gather(table, idx):
    info = plsc.get_sparse_core_info()
    nw = info.num_cores * info.num_subcores
    B = idx.shape[0]
    D = table.shape[1]
    bpw = B // nw
    mesh = plsc.VectorSubcoreMesh(core_axis_name="c", subcore_axis_name="s")

    @functools.partial(
        pl.kernel, mesh=mesh,
        out_type=jax.ShapeDtypeStruct((B, D), table.dtype),
        scratch_types=[
            pltpu.VMEM((bpw,), jnp.int32),
            pltpu.VMEM((bpw, D), jnp.float32),
            pltpu.SemaphoreType.DMA,
        ],
    )
    def gk(table_hbm, idx_hbm, out_hbm, idx_v, rows_v, sem):
        wid = lax.axis_index("s") * info.num_cores + lax.axis_index("c")
        base = wid * bpw
        pltpu.sync_copy(idx_hbm.at[pl.ds(base, bpw)], idx_v)
        pltpu.async_copy(table_hbm.at[idx_v], rows_v, sem).wait()
        pltpu.sync_copy(rows_v, out_hbm.at[pl.ds(base, bpw)])

    return gk(table, idx)


def kernel(x, codebook):
    K = x.shape[-1]
    xf = x.reshape(-1, K)
    x2 = jnp.sum(xf * xf, axis=1)[None, :]
    encoding, loss = _dist_argmin(xf, codebook, x2)
    nearest = _sc_gather(codebook, encoding)
    return encoding, loss, loss, nearest
